# Initial kernel scaffold; baseline (speedup 1.0000x reference)
import jax
import jax.numpy as jnp
from jax.experimental import pallas as pl


def kernel(q_coords, k_coords, bias):
    raise NotImplementedError("write your pallas kernel here")


# speedup vs baseline: 54.6272x; 54.6272x over previous
import functools

import jax
import jax.numpy as jnp
from jax import lax
from jax.experimental import pallas as pl
from jax.experimental.pallas import tpu as pltpu
from jax.experimental.pallas import tpu_sc as plsc

H = 32
W = 32
NH = 16
M = 1024
N = 1024
TH = 2 * H - 1
TW = 2 * W - 1
TSZ = TH * TW

NC = 2
NS = 16
L = 16
NW = NC * NS
ROWS = M // NW
CHUNKS = N // L


def _body(qy_h, qx_h, ky_h, kx_h, tab_h, out_h,
          tab_v, ky_v, kx_v, qy_v, qx_v, buf0, buf1, sem0, sem1):
    wid = lax.axis_index("s") * NC + lax.axis_index("c")
    base = wid * ROWS

    pltpu.sync_copy(tab_h, tab_v)
    pltpu.sync_copy(ky_h, ky_v)
    pltpu.sync_copy(kx_h, kx_v)
    pltpu.sync_copy(qy_h.at[pl.ds(base, ROWS)], qy_v)
    pltpu.sync_copy(qx_h.at[pl.ds(base, ROWS)], qx_v)

    bufs = (buf0, buf1)
    sems = (sem0, sem1)

    def row_pair(i, carry):
        r0 = i * 2
        for b in range(2):
            buf = bufs[b]
            sem = sems[b]
            r = r0 + b
            m = base + r

            @pl.when(r0 >= 2)
            def _():
                for h in range(NH):
                    pltpu.make_async_copy(buf.at[h], out_h.at[h, 0], sem).wait()

            qy = qy_v[r] + (H - 1)
            qx = qx_v[r] + (W - 1)

            def chunk(c, inner):
                off = c * L
                ky = ky_v[pl.ds(off, L)]
                kx = kx_v[pl.ds(off, L)]
                iy = jnp.clip(qy - ky, 0, 2 * H - 2)
                ix = jnp.clip(qx - kx, 0, 2 * W - 2)
                flat = iy * TW + ix
                for h in range(NH):
                    vals = plsc.load_gather(tab_v, [flat + h * TSZ])
                    buf[h, pl.ds(off, L)] = vals
                return inner

            lax.fori_loop(0, CHUNKS, chunk, 0)

            for h in range(NH):
                pltpu.async_copy(buf.at[h], out_h.at[h, m], sem)
        return carry

    lax.fori_loop(0, ROWS // 2, row_pair, 0)

    for b in range(2):
        for h in range(NH):
            pltpu.make_async_copy(bufs[b].at[h], out_h.at[h, 0], sems[b]).wait()


_sc_call = functools.partial(
    pl.kernel,
    out_type=jax.ShapeDtypeStruct((NH, M, N), jnp.float32),
    mesh=plsc.VectorSubcoreMesh(core_axis_name="c", subcore_axis_name="s"),
    compiler_params=pltpu.CompilerParams(needs_layout_passes=False),
    scratch_types=[
        pltpu.VMEM((NH * TSZ,), jnp.float32),
        pltpu.VMEM((N,), jnp.int32),
        pltpu.VMEM((N,), jnp.int32),
        pltpu.VMEM((ROWS, L), jnp.int32),
        pltpu.VMEM((ROWS, L), jnp.int32),
        pltpu.VMEM((NH, N), jnp.float32),
        pltpu.VMEM((NH, N), jnp.float32),
        pltpu.SemaphoreType.DMA,
        pltpu.SemaphoreType.DMA,
    ],
)(_body)


@jax.jit
def kernel(q_coords, k_coords, bias):
    qy = jnp.broadcast_to(q_coords[:, 0:1].astype(jnp.int32), (M, L))
    qx = jnp.broadcast_to(q_coords[:, 1:2].astype(jnp.int32), (M, L))
    ky = k_coords[:, 0].astype(jnp.int32)
    kx = k_coords[:, 1].astype(jnp.int32)
    tab = bias.reshape(NH * TSZ).astype(jnp.float32)
    return _sc_call(qy, qx, ky, kx, tab)


# speedup vs baseline: 141.3436x; 2.5874x over previous
import functools

import jax
import jax.numpy as jnp
from jax import lax
from jax.experimental import pallas as pl
from jax.experimental.pallas import tpu as pltpu
from jax.experimental.pallas import tpu_sc as plsc

H = 32
W = 32
NH = 16
M = 1024
N = 1024
TH = 2 * H - 1
TW = 2 * W - 1
TSZ = TH * TW

NC = 2
NS = 16
L = 16
NW = NC * NS
ROWS = M // NW
CHUNKS = N // L


def _body(qy_h, qx_h, ky_h, kx_h, tab_h, out_h,
          tab_v, ky_v, kx_v, qy_v, qx_v, buf0, buf1, sem0, sem1):
    wid = lax.axis_index("s") * NC + lax.axis_index("c")
    base = wid * ROWS

    pltpu.sync_copy(tab_h, tab_v)
    pltpu.sync_copy(ky_h, ky_v)
    pltpu.sync_copy(kx_h, kx_v)
    pltpu.sync_copy(qy_h.at[pl.ds(base, ROWS)], qy_v)
    pltpu.sync_copy(qx_h.at[pl.ds(base, ROWS)], qx_v)

    bufs = (buf0, buf1)
    sems = (sem0, sem1)

    def row_pair(i, carry):
        r0 = i * 2
        for b in range(2):
            buf = bufs[b]
            sem = sems[b]
            r = r0 + b
            m = base + r

            @pl.when(r0 >= 2)
            def _():
                for h in range(NH):
                    pltpu.make_async_copy(buf.at[h], out_h.at[h, 0], sem).wait()

            qy = qy_v[r] + (H - 1)
            qx = qx_v[r] + (W - 1)

            @plsc.parallel_loop(0, N, step=L, unroll=2)
            def chunk(off):
                ky = ky_v[pl.ds(off, L)]
                kx = kx_v[pl.ds(off, L)]
                iy = jnp.clip(qy - ky, 0, 2 * H - 2)
                ix = jnp.clip(qx - kx, 0, 2 * W - 2)
                flat = iy * TW + ix
                for h in range(NH):
                    vals = plsc.load_gather(tab_v, [flat + h * TSZ])
                    buf[h, pl.ds(off, L)] = vals

            for h in range(NH):
                pltpu.async_copy(buf.at[h], out_h.at[h, m], sem)
        return carry

    lax.fori_loop(0, ROWS // 2, row_pair, 0)

    for b in range(2):
        for h in range(NH):
            pltpu.make_async_copy(bufs[b].at[h], out_h.at[h, 0], sems[b]).wait()


_sc_call = functools.partial(
    pl.kernel,
    out_type=jax.ShapeDtypeStruct((NH, M, N), jnp.float32),
    mesh=plsc.VectorSubcoreMesh(core_axis_name="c", subcore_axis_name="s"),
    compiler_params=pltpu.CompilerParams(needs_layout_passes=False),
    scratch_types=[
        pltpu.VMEM((NH * TSZ,), jnp.float32),
        pltpu.VMEM((N,), jnp.int32),
        pltpu.VMEM((N,), jnp.int32),
        pltpu.VMEM((ROWS, L), jnp.int32),
        pltpu.VMEM((ROWS, L), jnp.int32),
        pltpu.VMEM((NH, N), jnp.float32),
        pltpu.VMEM((NH, N), jnp.float32),
        pltpu.SemaphoreType.DMA,
        pltpu.SemaphoreType.DMA,
    ],
)(_body)


@jax.jit
def kernel(q_coords, k_coords, bias):
    qy = jnp.broadcast_to(q_coords[:, 0:1].astype(jnp.int32), (M, L))
    qx = jnp.broadcast_to(q_coords[:, 1:2].astype(jnp.int32), (M, L))
    ky = k_coords[:, 0].astype(jnp.int32)
    kx = k_coords[:, 1].astype(jnp.int32)
    tab = bias.reshape(NH * TSZ).astype(jnp.float32)
    return _sc_call(qy, qx, ky, kx, tab)
